# baseline (speedup 1.0000x reference)
import functools

import jax
import jax.numpy as jnp
from jax import lax
from jax.experimental import pallas as pl
from jax.experimental.pallas import tpu as pltpu
from jax.experimental.pallas import tpu_sc as plsc

N = 10000
E = 320000
D = 128
H = 128
C = 40
L = 4
G = 64

NC = 2
NS = 16
NW = NC * NS
CHUNK = 128
NCH = 80
NPH = 2
PCH = NCH // NPH
EPW = NCH * CHUNK
E_PAD = EPW * NW
N_PAD = 10112
ZROWS = N_PAD // NS

_mesh = plsc.VectorSubcoreMesh(
    core_axis_name="c", subcore_axis_name="s", num_cores=NC, num_subcores=NS)


def _agg_body(h_hbm, src_hbm, dst_hbm, zeros_hbm, out_hbm,
              idx_s, idx_d, rows, rows2, agg_sh, semA, semB):
    c = lax.axis_index("c")
    s = lax.axis_index("s")
    w = s * NC + c

    pltpu.sync_copy(zeros_hbm.at[pl.ds(s * ZROWS, ZROWS)],
                    agg_sh.at[pl.ds(s * ZROWS, ZROWS)])
    plsc.subcore_barrier()

    for p in range(NPH):
        pltpu.sync_copy(src_hbm.at[w, pl.ds(p * PCH, PCH)], idx_s)
        pltpu.sync_copy(dst_hbm.at[w, pl.ds(p * PCH, PCH)], idx_d)

        @pl.loop(0, PCH, step=2)
        def _edge_chunk(j):
            dA = pltpu.async_copy(h_hbm.at[idx_s.at[j]], rows, semA)
            dB = pltpu.async_copy(h_hbm.at[idx_s.at[j + 1]], rows2, semB)
            dA.wait()
            pltpu.sync_copy(rows, agg_sh.at[idx_d.at[j]], add=True)
            dB.wait()
            pltpu.sync_copy(rows2, agg_sh.at[idx_d.at[j + 1]], add=True)

    plsc.subcore_barrier()
    pltpu.sync_copy(agg_sh.at[pl.ds(s * ZROWS, ZROWS)],
                    out_hbm.at[c, pl.ds(s * ZROWS, ZROWS)])


_agg = pl.kernel(
    _agg_body,
    out_type=jax.ShapeDtypeStruct((NC, N_PAD, H), jnp.float32),
    mesh=_mesh,
    scratch_types=[
        pltpu.VMEM((PCH, CHUNK), jnp.int32),
        pltpu.VMEM((PCH, CHUNK), jnp.int32),
        pltpu.VMEM((CHUNK, H), jnp.float32),
        pltpu.VMEM((CHUNK, H), jnp.float32),
        pltpu.VMEM_SHARED((N_PAD, H), jnp.float32),
        pltpu.SemaphoreType.DMA,
        pltpu.SemaphoreType.DMA,
    ],
)


def _emb_body(x_ref, w_ref, b_ref, out_ref):
    out_ref[...] = (jnp.dot(x_ref[...], w_ref[...],
                            preferred_element_type=jnp.float32) + b_ref[...])


_emb = pl.pallas_call(
    _emb_body, out_shape=jax.ShapeDtypeStruct((N, H), jnp.float32))


def _bn(z, g, b):
    mu = jnp.mean(z, axis=0, keepdims=True)
    d = z - mu
    var = jnp.mean(d * d, axis=0, keepdims=True)
    return d * lax.rsqrt(var + 1e-5) * g + b


def _layer_body(h_ref, a0_ref, a1_ref, sc_ref, w1_ref, b1_ref, w2_ref, b2_ref,
                g1_ref, be1_ref, g2_ref, be2_ref, out_ref):
    h = h_ref[...]
    z = sc_ref[0, 0] * h + a0_ref[:N] + a1_ref[:N]
    z = jnp.dot(z, w1_ref[...], preferred_element_type=jnp.float32) + b1_ref[...]
    z = jnp.maximum(_bn(z, g1_ref[...], be1_ref[...]), 0.0)
    z = jnp.dot(z, w2_ref[...], preferred_element_type=jnp.float32) + b2_ref[...]
    z = jnp.maximum(_bn(z, g2_ref[...], be2_ref[...]), 0.0)
    out_ref[...] = h + z


_layer = pl.pallas_call(
    _layer_body, out_shape=jax.ShapeDtypeStruct((N, H), jnp.float32))


def _pool_body(h0_ref, h1_ref, h2_ref, h3_ref, h4_ref, batch_ref, wp_ref,
               bp_ref, out_ref):
    hs = (h0_ref, h1_ref, h2_ref, h3_ref, h4_ref)
    y = jnp.zeros((N, C), jnp.float32)
    for i in range(L + 1):
        y = y + jnp.dot(hs[i][...], wp_ref[i],
                        preferred_element_type=jnp.float32)
    gids = lax.broadcasted_iota(jnp.int32, (G, N), 0)
    onehot = (gids == jnp.broadcast_to(batch_ref[...], (G, N))).astype(jnp.float32)
    counts = jnp.sum(onehot, axis=1, keepdims=True)
    pooled = jnp.dot(onehot, y, preferred_element_type=jnp.float32)
    out_ref[...] = (pooled / jnp.maximum(counts, 1.0)
                    + jnp.sum(bp_ref[...], axis=0, keepdims=True))


_pool = pl.pallas_call(
    _pool_body, out_shape=jax.ShapeDtypeStruct((G, C), jnp.float32))


def kernel(x, edge_index, batch, W_emb, b_emb, eps, W1, b1, W2, b2,
           bn_mlp_g, bn_mlp_b, bn_g, bn_b, Wp, bp):
    pad = E_PAD - E
    src_p = jnp.concatenate(
        [edge_index[0], jnp.zeros((pad,), jnp.int32)]).reshape(NW, NCH, CHUNK)
    dst_p = jnp.concatenate(
        [edge_index[1], jnp.full((pad,), N, jnp.int32)]).reshape(NW, NCH, CHUNK)
    zeros = jnp.zeros((N_PAD, H), jnp.float32)
    batch2 = batch.reshape(1, N)

    h = _emb(x, W_emb, b_emb.reshape(1, H))
    hidden = [h]
    for l in range(L):
        parts = _agg(h, src_p, dst_p, zeros)
        scale = (1.0 + eps[l]).reshape(1, 1)
        h = _layer(h, parts[0], parts[1], scale,
                   W1[l], b1[l].reshape(1, H), W2[l], b2[l].reshape(1, H),
                   bn_mlp_g[l].reshape(1, H), bn_mlp_b[l].reshape(1, H),
                   bn_g[l].reshape(1, H), bn_b[l].reshape(1, H))
        hidden.append(h)

    return _pool(hidden[0], hidden[1], hidden[2], hidden[3], hidden[4],
                 batch2, Wp, bp)


# speedup vs baseline: 1.0001x; 1.0001x over previous
import functools

import jax
import jax.numpy as jnp
from jax import lax
from jax.experimental import pallas as pl
from jax.experimental.pallas import tpu as pltpu
from jax.experimental.pallas import tpu_sc as plsc

N = 10000
E = 320000
D = 128
H = 128
C = 40
L = 4
G = 64

NC = 2
NS = 16
NW = NC * NS
CHUNK = 128
NCH = 80
NPH = 2
PCH = NCH // NPH
EPW = NCH * CHUNK
E_PAD = EPW * NW
N_PAD = 10112
ZROWS = N_PAD // NS

_mesh = plsc.VectorSubcoreMesh(
    core_axis_name="c", subcore_axis_name="s", num_cores=NC, num_subcores=NS)


def _agg_body(h_hbm, src_hbm, dst_hbm, zeros_hbm, out_hbm,
              idx_s, idx_d, rows, rows2, agg_sh, semA, semB):
    c = lax.axis_index("c")
    s = lax.axis_index("s")
    w = s * NC + c

    pltpu.sync_copy(zeros_hbm.at[pl.ds(s * ZROWS, ZROWS)],
                    agg_sh.at[pl.ds(s * ZROWS, ZROWS)])
    plsc.subcore_barrier()

    for p in range(NPH):
        pltpu.sync_copy(src_hbm.at[w, p], idx_s)
        pltpu.sync_copy(dst_hbm.at[w, p], idx_d)

        @pl.loop(0, PCH, step=2)
        def _edge_chunk(j):
            dA = pltpu.async_copy(h_hbm.at[idx_s.at[j]], rows, semA)
            dB = pltpu.async_copy(h_hbm.at[idx_s.at[j + 1]], rows2, semB)
            dA.wait()
            pltpu.sync_copy(rows, agg_sh.at[idx_d.at[j]], add=True)
            dB.wait()
            pltpu.sync_copy(rows2, agg_sh.at[idx_d.at[j + 1]], add=True)

    plsc.subcore_barrier()
    pltpu.sync_copy(agg_sh.at[pl.ds(s * ZROWS, ZROWS)],
                    out_hbm.at[c, pl.ds(s * ZROWS, ZROWS)])


_agg = pl.kernel(
    _agg_body,
    out_type=jax.ShapeDtypeStruct((NC, N_PAD, H), jnp.float32),
    mesh=_mesh,
    scratch_types=[
        pltpu.VMEM((PCH, CHUNK), jnp.int32),
        pltpu.VMEM((PCH, CHUNK), jnp.int32),
        pltpu.VMEM((CHUNK, H), jnp.float32),
        pltpu.VMEM((CHUNK, H), jnp.float32),
        pltpu.VMEM_SHARED((N_PAD, H), jnp.float32),
        pltpu.SemaphoreType.DMA,
        pltpu.SemaphoreType.DMA,
    ],
)


def _emb_body(x_ref, w_ref, b_ref, out_ref):
    out_ref[...] = (jnp.dot(x_ref[...], w_ref[...],
                            preferred_element_type=jnp.float32) + b_ref[...])


_emb = pl.pallas_call(
    _emb_body, out_shape=jax.ShapeDtypeStruct((N, H), jnp.float32))


def _bn(z, g, b):
    mu = jnp.mean(z, axis=0, keepdims=True)
    d = z - mu
    var = jnp.mean(d * d, axis=0, keepdims=True)
    return d * lax.rsqrt(var + 1e-5) * g + b


def _layer_body(h_ref, a0_ref, a1_ref, sc_ref, w1_ref, b1_ref, w2_ref, b2_ref,
                g1_ref, be1_ref, g2_ref, be2_ref, out_ref):
    h = h_ref[...]
    z = sc_ref[0, 0] * h + a0_ref[:N] + a1_ref[:N]
    z = jnp.dot(z, w1_ref[...], preferred_element_type=jnp.float32) + b1_ref[...]
    z = jnp.maximum(_bn(z, g1_ref[...], be1_ref[...]), 0.0)
    z = jnp.dot(z, w2_ref[...], preferred_element_type=jnp.float32) + b2_ref[...]
    z = jnp.maximum(_bn(z, g2_ref[...], be2_ref[...]), 0.0)
    out_ref[...] = h + z


_layer = pl.pallas_call(
    _layer_body, out_shape=jax.ShapeDtypeStruct((N, H), jnp.float32))


def _pool_body(h0_ref, h1_ref, h2_ref, h3_ref, h4_ref, batch_ref, wp_ref,
               bp_ref, out_ref):
    hs = (h0_ref, h1_ref, h2_ref, h3_ref, h4_ref)
    y = jnp.zeros((N, C), jnp.float32)
    for i in range(L + 1):
        y = y + jnp.dot(hs[i][...], wp_ref[i],
                        preferred_element_type=jnp.float32)
    gids = lax.broadcasted_iota(jnp.int32, (G, N), 0)
    onehot = (gids == jnp.broadcast_to(batch_ref[...], (G, N))).astype(jnp.float32)
    counts = jnp.sum(onehot, axis=1, keepdims=True)
    pooled = jnp.dot(onehot, y, preferred_element_type=jnp.float32)
    out_ref[...] = (pooled / jnp.maximum(counts, 1.0)
                    + jnp.sum(bp_ref[...], axis=0, keepdims=True))


_pool = pl.pallas_call(
    _pool_body, out_shape=jax.ShapeDtypeStruct((G, C), jnp.float32))


def kernel(x, edge_index, batch, W_emb, b_emb, eps, W1, b1, W2, b2,
           bn_mlp_g, bn_mlp_b, bn_g, bn_b, Wp, bp):
    pad = E_PAD - E
    src_p = jnp.concatenate(
        [edge_index[0], jnp.zeros((pad,), jnp.int32)]).reshape(NW, NPH, PCH, CHUNK)
    dst_p = jnp.concatenate(
        [edge_index[1], jnp.full((pad,), N, jnp.int32)]).reshape(NW, NPH, PCH, CHUNK)
    zeros = jnp.zeros((N_PAD, H), jnp.float32)
    batch2 = batch.reshape(1, N)

    h = _emb(x, W_emb, b_emb.reshape(1, H))
    hidden = [h]
    for l in range(L):
        parts = _agg(h, src_p, dst_p, zeros)
        scale = (1.0 + eps[l]).reshape(1, 1)
        h = _layer(h, parts[0], parts[1], scale,
                   W1[l], b1[l].reshape(1, H), W2[l], b2[l].reshape(1, H),
                   bn_mlp_g[l].reshape(1, H), bn_mlp_b[l].reshape(1, H),
                   bn_g[l].reshape(1, H), bn_b[l].reshape(1, H))
        hidden.append(h)

    return _pool(hidden[0], hidden[1], hidden[2], hidden[3], hidden[4],
                 batch2, Wp, bp)


# speedup vs baseline: 2.0392x; 2.0390x over previous
import functools

import jax
import jax.numpy as jnp
from jax import lax
from jax.experimental import pallas as pl
from jax.experimental.pallas import tpu as pltpu
from jax.experimental.pallas import tpu_sc as plsc

N = 10000
E = 320000
D = 128
H = 128
C = 40
L = 4
G = 64

NC = 2
NS = 16
NW = NC * NS
CHUNK = 128
NCH_F = 100
NCH_S = 57
CF = 0
E_F = NS * NCH_F * CHUNK
E_S = NS * NCH_S * CHUNK
N_PAD = 10112
ZROWS = N_PAD // NS

_mesh = plsc.VectorSubcoreMesh(
    core_axis_name="c", subcore_axis_name="s", num_cores=NC, num_subcores=NS)


def _agg_body(h_hbm, src_hbm, dst_hbm, zeros_hbm, out_hbm,
              idx_s, idx_d, rows, agg_sh, sem):
    c = lax.axis_index("c")
    s = lax.axis_index("s")

    pltpu.sync_copy(zeros_hbm.at[pl.ds(s * ZROWS, ZROWS)],
                    agg_sh.at[pl.ds(s * ZROWS, ZROWS)])
    pltpu.sync_copy(src_hbm.at[s, c], idx_s)
    pltpu.sync_copy(dst_hbm.at[s, c], idx_d)
    plsc.subcore_barrier()

    n = jnp.where(c == CF, NCH_F, NCH_S)

    @pl.loop(0, n)
    def _edge_chunk(j):
        pltpu.async_copy(h_hbm.at[idx_s.at[j]], rows, sem).wait()
        pltpu.sync_copy(rows, agg_sh.at[idx_d.at[j]], add=True)

    plsc.subcore_barrier()
    pltpu.sync_copy(agg_sh.at[pl.ds(s * ZROWS, ZROWS)],
                    out_hbm.at[c, pl.ds(s * ZROWS, ZROWS)])


_agg = pl.kernel(
    _agg_body,
    out_type=jax.ShapeDtypeStruct((NC, N_PAD, H), jnp.float32),
    mesh=_mesh,
    scratch_types=[
        pltpu.VMEM((NCH_F, CHUNK), jnp.int32),
        pltpu.VMEM((NCH_F, CHUNK), jnp.int32),
        pltpu.VMEM((CHUNK, H), jnp.float32),
        pltpu.VMEM_SHARED((N_PAD, H), jnp.float32),
        pltpu.SemaphoreType.DMA,
    ],
)


def _emb_body(x_ref, w_ref, b_ref, out_ref):
    out_ref[...] = (jnp.dot(x_ref[...], w_ref[...],
                            preferred_element_type=jnp.float32) + b_ref[...])


_emb = pl.pallas_call(
    _emb_body, out_shape=jax.ShapeDtypeStruct((N, H), jnp.float32))


def _bn(z, g, b):
    mu = jnp.mean(z, axis=0, keepdims=True)
    d = z - mu
    var = jnp.mean(d * d, axis=0, keepdims=True)
    return d * lax.rsqrt(var + 1e-5) * g + b


def _layer_body(h_ref, a0_ref, a1_ref, sc_ref, w1_ref, b1_ref, w2_ref, b2_ref,
                g1_ref, be1_ref, g2_ref, be2_ref, out_ref):
    h = h_ref[...]
    z = sc_ref[0, 0] * h + a0_ref[:N] + a1_ref[:N]
    z = jnp.dot(z, w1_ref[...], preferred_element_type=jnp.float32) + b1_ref[...]
    z = jnp.maximum(_bn(z, g1_ref[...], be1_ref[...]), 0.0)
    z = jnp.dot(z, w2_ref[...], preferred_element_type=jnp.float32) + b2_ref[...]
    z = jnp.maximum(_bn(z, g2_ref[...], be2_ref[...]), 0.0)
    out_ref[...] = h + z


_layer = pl.pallas_call(
    _layer_body, out_shape=jax.ShapeDtypeStruct((N, H), jnp.float32))


def _pool_body(h0_ref, h1_ref, h2_ref, h3_ref, h4_ref, batch_ref, wp_ref,
               bp_ref, out_ref):
    hs = (h0_ref, h1_ref, h2_ref, h3_ref, h4_ref)
    y = jnp.zeros((N, C), jnp.float32)
    for i in range(L + 1):
        y = y + jnp.dot(hs[i][...], wp_ref[i],
                        preferred_element_type=jnp.float32)
    gids = lax.broadcasted_iota(jnp.int32, (G, N), 0)
    onehot = (gids == jnp.broadcast_to(batch_ref[...], (G, N))).astype(jnp.float32)
    counts = jnp.sum(onehot, axis=1, keepdims=True)
    pooled = jnp.dot(onehot, y, preferred_element_type=jnp.float32)
    out_ref[...] = (pooled / jnp.maximum(counts, 1.0)
                    + jnp.sum(bp_ref[...], axis=0, keepdims=True))


_pool = pl.pallas_call(
    _pool_body, out_shape=jax.ShapeDtypeStruct((G, C), jnp.float32))


def kernel(x, edge_index, batch, W_emb, b_emb, eps, W1, b1, W2, b2,
           bn_mlp_g, bn_mlp_b, bn_g, bn_b, Wp, bp):
    def _split(ei, fill):
        fast = ei[:E_F].reshape(NS, NCH_F, CHUNK)
        tail = jnp.concatenate(
            [ei[E_F:], jnp.full((E_F + E_S - E,), fill, jnp.int32)])
        slow = jnp.concatenate(
            [tail.reshape(NS, NCH_S, CHUNK),
             jnp.full((NS, NCH_F - NCH_S, CHUNK), fill, jnp.int32)], axis=1)
        pair = (fast, slow) if CF == 0 else (slow, fast)
        return jnp.stack(pair, axis=1)

    src_p = _split(edge_index[0], 0)
    dst_p = _split(edge_index[1], N)
    zeros = jnp.zeros((N_PAD, H), jnp.float32)
    batch2 = batch.reshape(1, N)

    h = _emb(x, W_emb, b_emb.reshape(1, H))
    hidden = [h]
    for l in range(L):
        parts = _agg(h, src_p, dst_p, zeros)
        scale = (1.0 + eps[l]).reshape(1, 1)
        h = _layer(h, parts[0], parts[1], scale,
                   W1[l], b1[l].reshape(1, H), W2[l], b2[l].reshape(1, H),
                   bn_mlp_g[l].reshape(1, H), bn_mlp_b[l].reshape(1, H),
                   bn_g[l].reshape(1, H), bn_b[l].reshape(1, H))
        hidden.append(h)

    return _pool(hidden[0], hidden[1], hidden[2], hidden[3], hidden[4],
                 batch2, Wp, bp)
